# Initial kernel scaffold; baseline (speedup 1.0000x reference)
import jax
import jax.numpy as jnp
from jax.experimental import pallas as pl


def kernel(x, table):
    raise NotImplementedError("write your pallas kernel here")


# speedup vs baseline: 2.1594x; 2.1594x over previous
import jax
import jax.numpy as jnp
from jax.experimental import pallas as pl


def _add_kernel(x_ref, t_ref, o_ref):
    o_ref[...] = x_ref[...] + t_ref[...][None, :, :]


def kernel(x, table):
    B, S, D = x.shape
    TS = 256
    grid = (S // TS,)
    return pl.pallas_call(
        _add_kernel,
        grid=grid,
        in_specs=[
            pl.BlockSpec((B, TS, D), lambda s: (0, s, 0)),
            pl.BlockSpec((TS, D), lambda s: (s, 0)),
        ],
        out_specs=pl.BlockSpec((B, TS, D), lambda s: (0, s, 0)),
        out_shape=jax.ShapeDtypeStruct((B, S, D), x.dtype),
    )(x, table[:S])


# speedup vs baseline: 2.1600x; 1.0003x over previous
import jax
import jax.numpy as jnp
from jax.experimental import pallas as pl


def _add_kernel(x_ref, t_ref, o_ref):
    o_ref[...] = x_ref[...] + t_ref[...][None, :, :]


def kernel(x, table):
    B, S, D = x.shape
    TS = 512
    grid = (S // TS,)
    return pl.pallas_call(
        _add_kernel,
        grid=grid,
        in_specs=[
            pl.BlockSpec((B, TS, D), lambda s: (0, s, 0)),
            pl.BlockSpec((TS, D), lambda s: (s, 0)),
        ],
        out_specs=pl.BlockSpec((B, TS, D), lambda s: (0, s, 0)),
        out_shape=jax.ShapeDtypeStruct((B, S, D), x.dtype),
    )(x, table[:S])
